# baseline (speedup 1.0000x reference)
import functools

import jax
import jax.numpy as jnp
from jax import lax
from jax.experimental import pallas as pl
from jax.experimental.pallas import tpu as pltpu
from jax.experimental.pallas import tpu_sc as plsc

D = 64
CHUNK = 128
NBUF = 4
SCALE = 8.0


def _scale_chunk(src, dst):
    def row(i, _):
        for k in range(D // 16):
            sl = pl.ds(k * 16, 16)
            dst[i, sl] = src[i, sl] * SCALE
        return 0

    lax.fori_loop(0, CHUNK, row, 0, unroll=4)


def _make_emb_call(n_tokens_flat):
    info = plsc.get_sparse_core_info()
    nw = info.num_cores * info.num_subcores
    assert n_tokens_flat % (nw * CHUNK) == 0
    steps = n_tokens_flat // (nw * CHUNK)
    groups = steps // NBUF
    assert groups >= 3 and steps % NBUF == 0

    mesh = plsc.VectorSubcoreMesh(core_axis_name="c", subcore_axis_name="s")

    @functools.partial(
        pl.kernel,
        out_type=jax.ShapeDtypeStruct((n_tokens_flat, D), jnp.float32),
        mesh=mesh,
        compiler_params=pltpu.CompilerParams(use_tc_tiling_on_sc=False),
        scratch_types=[
            pltpu.VMEM((steps, CHUNK), jnp.int32),
            [pltpu.VMEM((CHUNK, D), jnp.float32) for _ in range(NBUF)],
            [pltpu.VMEM((CHUNK, D), jnp.float32) for _ in range(NBUF)],
            [pltpu.SemaphoreType.DMA for _ in range(NBUF)],
            [pltpu.SemaphoreType.DMA for _ in range(NBUF)],
        ],
    )
    def emb(idx_hbm, table_hbm, out_hbm, idx_v, gbufs, sbufs, gsems, ssems):
        wid = lax.axis_index("s") * info.num_cores + lax.axis_index("c")
        idx_row0 = wid * steps
        out_row0 = wid * steps * CHUNK

        pltpu.sync_copy(idx_hbm.at[pl.ds(idx_row0, steps)], idx_v)

        def start_gather(b, s):
            pltpu.make_async_copy(
                table_hbm.at[idx_v.at[s]], gbufs[b], gsems[b]
            ).start()

        def wait_gather(b, s):
            pltpu.make_async_copy(
                table_hbm.at[idx_v.at[s]], gbufs[b], gsems[b]
            ).wait()

        def start_store(b, s):
            pltpu.make_async_copy(
                sbufs[b], out_hbm.at[pl.ds(out_row0 + s * CHUNK, CHUNK)],
                ssems[b],
            ).start()

        def wait_store(b, s):
            pltpu.make_async_copy(
                sbufs[b], out_hbm.at[pl.ds(out_row0 + s * CHUNK, CHUNK)],
                ssems[b],
            ).wait()

        for b in range(NBUF):
            start_gather(b, b)

        for b in range(NBUF):
            wait_gather(b, b)
            _scale_chunk(gbufs[b], sbufs[b])
            start_store(b, b)
            start_gather(b, NBUF + b)

        def group(g, _):
            for b in range(NBUF):
                s = g * NBUF + b
                wait_gather(b, s)
                wait_store(b, s - NBUF)
                _scale_chunk(gbufs[b], sbufs[b])
                start_store(b, s)
                start_gather(b, s + NBUF)
            return 0

        lax.fori_loop(1, groups - 1, group, 0)

        for b in range(NBUF):
            s = (groups - 1) * NBUF + b
            wait_gather(b, s)
            wait_store(b, s - NBUF)
            _scale_chunk(gbufs[b], sbufs[b])
            start_store(b, s)
        for b in range(NBUF):
            wait_store(b, (groups - 1) * NBUF + b)

    return emb


def kernel(inp_tokens, emb_table):
    bsz, seq = inp_tokens.shape
    n = bsz * seq
    idx = inp_tokens.reshape(n // CHUNK, CHUNK)
    out = _make_emb_call(n)(idx, emb_table)
    return out.reshape(bsz, seq, D)


# speedup vs baseline: 1.2700x; 1.2700x over previous
import functools

import jax
import jax.numpy as jnp
from jax import lax
from jax.experimental import pallas as pl
from jax.experimental.pallas import tpu as pltpu
from jax.experimental.pallas import tpu_sc as plsc

D = 64
CHUNK = 128
NBUF = 4
SCALE = 8.0


def _scale_chunk(src, dst):
    @plsc.parallel_loop(0, CHUNK, step=1, unroll=8)
    def _(i):
        for k in range(D // 16):
            sl = pl.ds(k * 16, 16)
            dst[i, sl] = src[i, sl] * SCALE


def _make_emb_call(n_tokens_flat):
    info = plsc.get_sparse_core_info()
    nw = info.num_cores * info.num_subcores
    assert n_tokens_flat % (nw * CHUNK) == 0
    steps = n_tokens_flat // (nw * CHUNK)
    groups = steps // NBUF
    assert groups >= 3 and steps % NBUF == 0

    mesh = plsc.VectorSubcoreMesh(core_axis_name="c", subcore_axis_name="s")

    @functools.partial(
        pl.kernel,
        out_type=jax.ShapeDtypeStruct((n_tokens_flat, D), jnp.float32),
        mesh=mesh,
        compiler_params=pltpu.CompilerParams(use_tc_tiling_on_sc=False),
        scratch_types=[
            pltpu.VMEM((steps, CHUNK), jnp.int32),
            [pltpu.VMEM((CHUNK, D), jnp.float32) for _ in range(NBUF)],
            [pltpu.VMEM((CHUNK, D), jnp.float32) for _ in range(NBUF)],
            [pltpu.SemaphoreType.DMA for _ in range(NBUF)],
            [pltpu.SemaphoreType.DMA for _ in range(NBUF)],
        ],
    )
    def emb(idx_hbm, table_hbm, out_hbm, idx_v, gbufs, sbufs, gsems, ssems):
        wid = lax.axis_index("s") * info.num_cores + lax.axis_index("c")
        idx_row0 = wid * steps
        out_row0 = wid * steps * CHUNK

        pltpu.sync_copy(idx_hbm.at[pl.ds(idx_row0, steps)], idx_v)

        def start_gather(b, s):
            pltpu.make_async_copy(
                table_hbm.at[idx_v.at[s]], gbufs[b], gsems[b]
            ).start()

        def wait_gather(b, s):
            pltpu.make_async_copy(
                table_hbm.at[idx_v.at[s]], gbufs[b], gsems[b]
            ).wait()

        def start_store(b, s):
            pltpu.make_async_copy(
                sbufs[b], out_hbm.at[pl.ds(out_row0 + s * CHUNK, CHUNK)],
                ssems[b],
            ).start()

        def wait_store(b, s):
            pltpu.make_async_copy(
                sbufs[b], out_hbm.at[pl.ds(out_row0 + s * CHUNK, CHUNK)],
                ssems[b],
            ).wait()

        for b in range(NBUF):
            start_gather(b, b)

        for b in range(NBUF):
            wait_gather(b, b)
            _scale_chunk(gbufs[b], sbufs[b])
            start_store(b, b)
            start_gather(b, NBUF + b)

        def group(g, _):
            for b in range(NBUF):
                s = g * NBUF + b
                wait_gather(b, s)
                wait_store(b, s - NBUF)
                _scale_chunk(gbufs[b], sbufs[b])
                start_store(b, s)
                start_gather(b, s + NBUF)
            return 0

        lax.fori_loop(1, groups - 1, group, 0)

        for b in range(NBUF):
            s = (groups - 1) * NBUF + b
            wait_gather(b, s)
            wait_store(b, s - NBUF)
            _scale_chunk(gbufs[b], sbufs[b])
            start_store(b, s)
        for b in range(NBUF):
            wait_store(b, (groups - 1) * NBUF + b)

    return emb


def kernel(inp_tokens, emb_table):
    bsz, seq = inp_tokens.shape
    n = bsz * seq
    idx = inp_tokens.reshape(n // CHUNK, CHUNK)
    out = _make_emb_call(n)(idx, emb_table)
    return out.reshape(bsz, seq, D)


# speedup vs baseline: 1.5353x; 1.2089x over previous
import functools

import jax
import jax.numpy as jnp
from jax import lax
from jax.experimental import pallas as pl
from jax.experimental.pallas import tpu as pltpu
from jax.experimental.pallas import tpu_sc as plsc

D = 64
CHUNK = 128
NBUF = 4
SCALE = 8.0


def _scale_chunk(src, dst):
    @plsc.parallel_loop(0, CHUNK, step=1, unroll=8)
    def _(i):
        for k in range(D // 16):
            sl = pl.ds(k * 16, 16)
            dst[i, sl] = src[i, sl] * SCALE


def _make_emb_call(n_tokens_flat):
    info = plsc.get_sparse_core_info()
    nw = info.num_cores * info.num_subcores
    assert n_tokens_flat % (nw * CHUNK) == 0
    steps = n_tokens_flat // (nw * CHUNK)
    groups = steps // NBUF
    assert groups >= 3 and steps % NBUF == 0

    mesh = plsc.VectorSubcoreMesh(core_axis_name="c", subcore_axis_name="s")

    @functools.partial(
        pl.kernel,
        out_type=jax.ShapeDtypeStruct((n_tokens_flat, 2 * D), jnp.float32),
        mesh=mesh,
        compiler_params=pltpu.CompilerParams(use_tc_tiling_on_sc=False),
        scratch_types=[
            pltpu.VMEM((steps, CHUNK), jnp.int32),
            [pltpu.VMEM((CHUNK, D), jnp.float32) for _ in range(NBUF)],
            [pltpu.VMEM((CHUNK, 2 * D), jnp.float32) for _ in range(NBUF)],
            [pltpu.SemaphoreType.DMA for _ in range(NBUF)],
            [pltpu.SemaphoreType.DMA for _ in range(NBUF)],
        ],
    )
    def emb(idx_hbm, table_hbm, out_hbm, idx_v, gbufs, sbufs, gsems, ssems):
        wid = lax.axis_index("s") * info.num_cores + lax.axis_index("c")
        idx_row0 = wid * steps
        out_row0 = wid * steps * CHUNK

        pltpu.sync_copy(idx_hbm.at[pl.ds(idx_row0, steps)], idx_v)

        def start_gather(b, s):
            pltpu.make_async_copy(
                table_hbm.at[idx_v.at[s]], gbufs[b], gsems[b]
            ).start()

        def wait_gather(b, s):
            pltpu.make_async_copy(
                table_hbm.at[idx_v.at[s]], gbufs[b], gsems[b]
            ).wait()

        def start_store(b, s):
            pltpu.make_async_copy(
                sbufs[b], out_hbm.at[pl.ds(out_row0 + s * CHUNK, CHUNK)],
                ssems[b],
            ).start()

        def wait_store(b, s):
            pltpu.make_async_copy(
                sbufs[b], out_hbm.at[pl.ds(out_row0 + s * CHUNK, CHUNK)],
                ssems[b],
            ).wait()

        for b in range(NBUF):
            start_gather(b, b)

        for b in range(NBUF):
            wait_gather(b, b)
            _scale_chunk(gbufs[b], sbufs[b])
            start_store(b, b)
            start_gather(b, NBUF + b)

        def group(g, _):
            for b in range(NBUF):
                s = g * NBUF + b
                wait_gather(b, s)
                wait_store(b, s - NBUF)
                _scale_chunk(gbufs[b], sbufs[b])
                start_store(b, s)
                start_gather(b, s + NBUF)
            return 0

        lax.fori_loop(1, groups - 1, group, 0)

        for b in range(NBUF):
            s = (groups - 1) * NBUF + b
            wait_gather(b, s)
            wait_store(b, s - NBUF)
            _scale_chunk(gbufs[b], sbufs[b])
            start_store(b, s)
        for b in range(NBUF):
            wait_store(b, (groups - 1) * NBUF + b)

    return emb


def kernel(inp_tokens, emb_table):
    bsz, seq = inp_tokens.shape
    n = bsz * seq
    idx = inp_tokens.reshape(n // CHUNK, CHUNK)
    out = _make_emb_call(n)(idx, emb_table)
    return out[:, :D].reshape(bsz, seq, D)


# speedup vs baseline: 1.6849x; 1.0975x over previous
import functools

import jax
import jax.numpy as jnp
from jax import lax
from jax.experimental import pallas as pl
from jax.experimental.pallas import tpu as pltpu
from jax.experimental.pallas import tpu_sc as plsc

D = 64
CHUNK = 128
NBUF = 4
SCALE = 8.0


def _scale_chunk(src, dst):
    @plsc.parallel_loop(0, CHUNK, step=1, unroll=8)
    def _(i):
        for k in range(D // 16):
            sl = pl.ds(k * 16, 16)
            dst[i, sl] = src[i, sl] * SCALE


def _make_emb_call(n_tokens_flat):
    info = plsc.get_sparse_core_info()
    nw = info.num_cores * info.num_subcores
    assert n_tokens_flat % (nw * CHUNK) == 0
    steps = n_tokens_flat // (nw * CHUNK)
    groups = steps // NBUF
    assert groups >= 3 and steps % NBUF == 0

    mesh = plsc.VectorSubcoreMesh(core_axis_name="c", subcore_axis_name="s")

    @functools.partial(
        pl.kernel,
        out_type=jax.ShapeDtypeStruct((n_tokens_flat, 2 * D), jnp.float32),
        mesh=mesh,
        compiler_params=pltpu.CompilerParams(use_tc_tiling_on_sc=False),
        scratch_types=[
            pltpu.VMEM((steps, CHUNK), jnp.int32),
            [pltpu.VMEM((CHUNK, D), jnp.float32) for _ in range(NBUF)],
            [pltpu.VMEM((CHUNK, D), jnp.float32) for _ in range(NBUF)],
            [pltpu.SemaphoreType.DMA for _ in range(NBUF)],
            [pltpu.SemaphoreType.DMA for _ in range(NBUF)],
        ],
    )
    def emb(idx_hbm, table_hbm, out_hbm, idx_v, gbufs, sbufs, gsems, ssems):
        wid = lax.axis_index("s") * info.num_cores + lax.axis_index("c")
        idx_row0 = wid * steps
        out_row0 = wid * steps * CHUNK

        pltpu.sync_copy(idx_hbm.at[pl.ds(idx_row0, steps)], idx_v)

        def start_gather(b, s):
            pltpu.make_async_copy(
                table_hbm.at[idx_v.at[s]], gbufs[b], gsems[b]
            ).start()

        def wait_gather(b, s):
            pltpu.make_async_copy(
                table_hbm.at[idx_v.at[s]], gbufs[b], gsems[b]
            ).wait()

        def start_store(b, s):
            pltpu.make_async_copy(
                sbufs[b],
                out_hbm.at[pl.ds(out_row0 + s * CHUNK, CHUNK), pl.ds(0, D)],
                ssems[b],
            ).start()

        def wait_store(b, s):
            pltpu.make_async_copy(
                sbufs[b],
                out_hbm.at[pl.ds(out_row0 + s * CHUNK, CHUNK), pl.ds(0, D)],
                ssems[b],
            ).wait()

        for b in range(NBUF):
            start_gather(b, b)

        for b in range(NBUF):
            wait_gather(b, b)
            _scale_chunk(gbufs[b], sbufs[b])
            start_store(b, b)
            start_gather(b, NBUF + b)

        def group(g, _):
            for b in range(NBUF):
                s = g * NBUF + b
                wait_gather(b, s)
                wait_store(b, s - NBUF)
                _scale_chunk(gbufs[b], sbufs[b])
                start_store(b, s)
                start_gather(b, s + NBUF)
            return 0

        lax.fori_loop(1, groups - 1, group, 0)

        for b in range(NBUF):
            s = (groups - 1) * NBUF + b
            wait_gather(b, s)
            wait_store(b, s - NBUF)
            _scale_chunk(gbufs[b], sbufs[b])
            start_store(b, s)
        for b in range(NBUF):
            wait_store(b, (groups - 1) * NBUF + b)

    return emb


def kernel(inp_tokens, emb_table):
    bsz, seq = inp_tokens.shape
    n = bsz * seq
    idx = inp_tokens.reshape(n // CHUNK, CHUNK)
    out = _make_emb_call(n)(idx, emb_table)
    return out[:, :D].reshape(bsz, seq, D)
